# baseline (speedup 1.0000x reference)
import functools

import jax
import jax.numpy as jnp
from jax import lax
from jax.experimental import pallas as pl
from jax.experimental.pallas import tpu as pltpu
from jax.experimental.pallas import tpu_sc as plsc

_CHUNK_ROWS = 32
_NBUF = 2


def _make_sc_broadcast(batch: int, seq_len: int, d: int):
    info = plsc.get_sparse_core_info()
    nc, ns = info.num_cores, info.num_subcores
    nw = nc * ns
    assert seq_len % nw == 0
    rows_per_w = seq_len // nw
    r = min(_CHUNK_ROWS, rows_per_w)
    assert rows_per_w % r == 0
    nchunks = rows_per_w // r
    nbuf = min(_NBUF, nchunks)

    mesh = plsc.VectorSubcoreMesh(core_axis_name="c", subcore_axis_name="s")

    assert nbuf == 2 and nchunks >= 4 and nchunks % 2 == 0

    @functools.partial(
        pl.kernel,
        mesh=mesh,
        out_type=jax.ShapeDtypeStruct((batch, seq_len, d), jnp.float32),
        scratch_types=(
            [pltpu.VMEM((r, d), jnp.float32)] * nbuf
            + [pltpu.SemaphoreType.DMA] * (2 * nbuf)
        ),
    )
    def k(pe_hbm, out_hbm, buf0, buf1, g0, g1, s0, s1):
        bufs, gsem, ssem = [buf0, buf1], [g0, g1], [s0, s1]
        wid = lax.axis_index("s") * nc + lax.axis_index("c")
        base = wid * rows_per_w

        def gather(c, par):
            return pltpu.make_async_copy(
                pe_hbm.at[pl.ds(base + c * r, r)], bufs[par], gsem[par])

        def scatters(c, par):
            return [
                pltpu.make_async_copy(
                    bufs[par], out_hbm.at[b, pl.ds(base + c * r, r)], ssem[par])
                for b in range(batch)
            ]

        gather(0, 0).start()
        gather(0, 0).wait()
        gather(1, 1).start()
        for cp in scatters(0, 0):
            cp.start()

        @pl.loop(1, nchunks - 1, step=2)
        def _steady(c):
            for j in range(2):
                par = (1 + j) % 2
                cc = c + j
                gather(cc, par).wait()
                for cp in scatters(cc - 1, 1 - par):
                    cp.wait()
                gather(cc + 1, 1 - par).start()
                for cp in scatters(cc, par):
                    cp.start()

        last = nchunks - 1
        par = last % 2
        gather(last, par).wait()
        for cp in scatters(last - 1, 1 - par):
            cp.wait()
        for cp in scatters(last, par):
            cp.start()
        for cp in scatters(last, par):
            cp.wait()

    return k


def kernel(x, pos_embed):
    batch, seq_len = x.shape[0], x.shape[1]
    d = pos_embed.shape[1]
    return _make_sc_broadcast(batch, seq_len, d)(pos_embed)
